# baseline (speedup 1.0000x reference)
import functools
import jax
import jax.numpy as jnp
from jax import lax
from jax.experimental import pallas as pl
from jax.experimental.pallas import tpu as pltpu
from jax.experimental.pallas import tpu_sc as plsc

_NC, _NS, _L = 2, 16, 16
_NW = _NC * _NS
_ROWS, _COLS = 16384, 200
_RPW = _ROWS // _NW
_CH = 128
_K = _RPW // _CH

_COL_OFFS = list(range(0, _COLS - _L + 1, _L))
if _COL_OFFS[-1] != _COLS - _L:
    _COL_OFFS.append(_COLS - _L)


def _sc_body(x_hbm, v_hbm, rv_hbm, out_hbm, buf0, buf1, vv, rvv,
             in_sem0, in_sem1, out_sem0, out_sem1):
    bufs = (buf0, buf1)
    in_sems = (in_sem0, in_sem1)
    out_sems = (out_sem0, out_sem1)
    wid = lax.axis_index("s") * _NC + lax.axis_index("c")
    base = wid * _RPW
    pltpu.sync_copy(v_hbm, vv)
    pltpu.sync_copy(rv_hbm, rvv)
    v = vv[...]
    rv = rvv[...]

    in_handles = [None] * _K
    out_handles = [None] * _K

    def start_in(k):
        b = k % 2
        in_handles[k] = pltpu.async_copy(
            x_hbm.at[pl.ds(base + k * _CH, _CH)], bufs[b], in_sems[b])

    def start_out(k):
        b = k % 2
        out_handles[k] = pltpu.async_copy(
            bufs[b], out_hbm.at[pl.ds(base + k * _CH, _CH)], out_sems[b])

    start_in(0)
    for k in range(_K):
        b = k % 2
        if k + 1 < _K:
            if k >= 1:
                out_handles[k - 1].wait()
            start_in(k + 1)
        in_handles[k].wait()
        buf = bufs[b]

        def row_body(r_i, carry):
            for c in _COL_OFFS:
                x = buf[r_i, pl.ds(c, _L)]
                q = (x.astype(jnp.float32) * rv).astype(jnp.int32)
                r = x - q * v
                r = jnp.where(r < 0, r + v, r)
                r = jnp.where(r >= v, r - v, r)
                buf[r_i, pl.ds(c, _L)] = r
            return carry

        lax.fori_loop(0, _CH, row_body, 0)
        start_out(k)
    out_handles[_K - 2].wait()
    out_handles[_K - 1].wait()


def kernel(indices, feature_idx, vocab_sizes):
    vocab = jax.lax.dynamic_index_in_dim(vocab_sizes, feature_idx, keepdims=False)
    vv = jnp.full((_L,), vocab, dtype=jnp.int32)
    rvv = jnp.full((_L,), 1.0 / vocab.astype(jnp.float32), dtype=jnp.float32)
    mesh = plsc.VectorSubcoreMesh(
        core_axis_name="c", subcore_axis_name="s", num_cores=_NC, num_subcores=_NS
    )
    f = functools.partial(
        pl.kernel,
        compiler_params=pltpu.CompilerParams(use_tc_tiling_on_sc=True),
        out_type=jax.ShapeDtypeStruct((_ROWS, _COLS), indices.dtype),
        mesh=mesh,
        scratch_types=[
            pltpu.VMEM((_CH, _COLS), jnp.int32),
            pltpu.VMEM((_CH, _COLS), jnp.int32),
            pltpu.VMEM((_L,), jnp.int32),
            pltpu.VMEM((_L,), jnp.float32),
            pltpu.SemaphoreType.DMA,
            pltpu.SemaphoreType.DMA,
            pltpu.SemaphoreType.DMA,
            pltpu.SemaphoreType.DMA,
        ],
    )(_sc_body)
    return f(indices, vv, rvv)


# speedup vs baseline: 5.2673x; 5.2673x over previous
import jax
import jax.numpy as jnp
from jax.experimental import pallas as pl
from jax.experimental.pallas import tpu as pltpu

_ROWS, _COLS = 16384, 200
_BLK = 2048


def _mod_body(v_ref, rv_ref, x_ref, o_ref):
    v = v_ref[0]
    rv = rv_ref[0]
    x = x_ref[...]
    q = jnp.floor(x.astype(jnp.float32) * rv).astype(jnp.int32)
    r = x - q * v
    r = jnp.where(r < 0, r + v, r)
    r = jnp.where(r >= v, r - v, r)
    o_ref[...] = r


def kernel(indices, feature_idx, vocab_sizes):
    vocab = jax.lax.dynamic_index_in_dim(vocab_sizes, feature_idx, keepdims=True)
    recip = 1.0 / vocab.astype(jnp.float32)
    xt = jnp.swapaxes(indices, 0, 1)
    grid = (_ROWS // _BLK,)
    out_t = pl.pallas_call(
        _mod_body,
        grid=grid,
        in_specs=[
            pl.BlockSpec(memory_space=pltpu.SMEM),
            pl.BlockSpec(memory_space=pltpu.SMEM),
            pl.BlockSpec((_COLS, _BLK), lambda i: (0, i)),
        ],
        out_specs=pl.BlockSpec((_COLS, _BLK), lambda i: (0, i)),
        out_shape=jax.ShapeDtypeStruct((_COLS, _ROWS), indices.dtype),
    )(vocab, recip, xt)
    return jnp.swapaxes(out_t, 0, 1)


# speedup vs baseline: 5.5986x; 1.0629x over previous
import jax
import jax.numpy as jnp
from jax.experimental import pallas as pl
from jax.experimental.pallas import tpu as pltpu

_ROWS, _COLS = 16384, 200
_BLK = 2048


def _mod_body(fi_ref, vs_ref, x_ref, o_ref):
    v = vs_ref[fi_ref[0]]
    rv = 1.0 / v.astype(jnp.float32)
    x = x_ref[...]
    q = jnp.floor(x.astype(jnp.float32) * rv).astype(jnp.int32)
    r = x - q * v
    r = jnp.where(r < 0, r + v, r)
    r = jnp.where(r >= v, r - v, r)
    o_ref[...] = r


def kernel(indices, feature_idx, vocab_sizes):
    fi = jnp.reshape(jnp.asarray(feature_idx, dtype=jnp.int32), (1,))
    xt = jnp.swapaxes(indices, 0, 1)
    grid = (_ROWS // _BLK,)
    out_t = pl.pallas_call(
        _mod_body,
        grid=grid,
        in_specs=[
            pl.BlockSpec(memory_space=pltpu.SMEM),
            pl.BlockSpec(memory_space=pltpu.SMEM),
            pl.BlockSpec((_COLS, _BLK), lambda i: (0, i)),
        ],
        out_specs=pl.BlockSpec((_COLS, _BLK), lambda i: (0, i)),
        out_shape=jax.ShapeDtypeStruct((_COLS, _ROWS), indices.dtype),
    )(fi, vocab_sizes, xt)
    return jnp.swapaxes(out_t, 0, 1)


# speedup vs baseline: 5.8728x; 1.0490x over previous
import jax
import jax.numpy as jnp
from jax.experimental import pallas as pl
from jax.experimental.pallas import tpu as pltpu

_ROWS, _COLS = 16384, 200
_BLK = 4096


def _mod_body(fi_ref, vs_ref, x_ref, o_ref):
    v = vs_ref[fi_ref[0]]
    rv = 1.0 / v.astype(jnp.float32)
    x = x_ref[...]
    q = jnp.floor(x.astype(jnp.float32) * rv).astype(jnp.int32)
    r = x - q * v
    r = jnp.where(r < 0, r + v, r)
    r = jnp.where(r >= v, r - v, r)
    o_ref[...] = r


def kernel(indices, feature_idx, vocab_sizes):
    fi = jnp.reshape(jnp.asarray(feature_idx, dtype=jnp.int32), (1,))
    xt = jnp.swapaxes(indices, 0, 1)
    grid = (_ROWS // _BLK,)
    out_t = pl.pallas_call(
        _mod_body,
        grid=grid,
        in_specs=[
            pl.BlockSpec(memory_space=pltpu.SMEM),
            pl.BlockSpec(memory_space=pltpu.SMEM),
            pl.BlockSpec((_COLS, _BLK), lambda i: (0, i)),
        ],
        out_specs=pl.BlockSpec((_COLS, _BLK), lambda i: (0, i)),
        out_shape=jax.ShapeDtypeStruct((_COLS, _ROWS), indices.dtype),
    )(fi, vocab_sizes, xt)
    return jnp.swapaxes(out_t, 0, 1)
